# baseline (speedup 1.0000x reference)
import functools

import jax
import jax.numpy as jnp
from jax import lax
from jax.experimental import pallas as pl
from jax.experimental.pallas import tpu as pltpu
from jax.experimental.pallas import tpu_sc as plsc

VOCAB = 100000
D = 128
SEQ = 20
N_TOK = 640
FC_IN = N_TOK * D
FC_OUT = 32

N_WORKERS = 16
ROWS_PER_W = N_TOK // N_WORKERS

K_BLK = 16384
N_BLKS = FC_IN // K_BLK


def _sc_gather_body(table_hbm, idx_hbm, out_hbm, idx_v, rows_v, sem):
    wid = lax.axis_index("s")
    base = wid * ROWS_PER_W
    pltpu.sync_copy(idx_hbm.at[pl.ds(base, ROWS_PER_W)], idx_v)
    pltpu.async_copy(table_hbm.at[idx_v], rows_v, sem).wait()
    pltpu.sync_copy(rows_v, out_hbm.at[pl.ds(base, ROWS_PER_W)])


_sc_gather = functools.partial(
    pl.kernel,
    mesh=plsc.VectorSubcoreMesh(
        core_axis_name="c", subcore_axis_name="s", num_cores=1
    ),
    out_type=jax.ShapeDtypeStruct((N_TOK, D), jnp.float32),
    scratch_types=[
        pltpu.VMEM((ROWS_PER_W,), jnp.int32),
        pltpu.VMEM((ROWS_PER_W, D), jnp.float32),
        pltpu.SemaphoreType.DMA,
    ],
)(_sc_gather_body)


def _tc_matvec_body(w_ref, x_ref, b_ref, o_ref):
    i = pl.program_id(0)
    part = jnp.sum(w_ref[...] * x_ref[...], axis=1)

    @pl.when(i == 0)
    def _():
        o_ref[...] = part + b_ref[...]

    @pl.when(i > 0)
    def _():
        o_ref[...] = o_ref[...] + part

    @pl.when(i == N_BLKS - 1)
    def _():
        acc = o_ref[...]
        o_ref[...] = 1.0 / (1.0 + jnp.exp(-acc))


_tc_matvec = pl.pallas_call(
    _tc_matvec_body,
    grid=(N_BLKS,),
    in_specs=[
        pl.BlockSpec((FC_OUT, K_BLK), lambda i: (0, i)),
        pl.BlockSpec((1, K_BLK), lambda i: (0, i)),
        pl.BlockSpec((FC_OUT,), lambda i: (0,)),
    ],
    out_specs=pl.BlockSpec((FC_OUT,), lambda i: (0,)),
    out_shape=jax.ShapeDtypeStruct((FC_OUT,), jnp.float32),
)


@jax.jit
def kernel(text, emb_table, W, b):
    idx = text.reshape(-1)
    flat = _sc_gather(emb_table, idx)
    x = flat.reshape(1, FC_IN)
    return _tc_matvec(W, x, b)


# speedup vs baseline: 1.0678x; 1.0678x over previous
import functools

import jax
import jax.numpy as jnp
from jax import lax
from jax.experimental import pallas as pl
from jax.experimental.pallas import tpu as pltpu
from jax.experimental.pallas import tpu_sc as plsc

VOCAB = 100000
D = 128
SEQ = 20
N_TOK = 640
FC_IN = N_TOK * D
FC_OUT = 32

N_WORKERS = 16
ROWS_PER_W = N_TOK // N_WORKERS

K_BLK = 40960
N_BLKS = FC_IN // K_BLK


def _sc_gather_body(table_hbm, idx_hbm, out_hbm, idx_v, rows_v, sem):
    wid = lax.axis_index("s")
    base = wid * ROWS_PER_W
    pltpu.sync_copy(idx_hbm.at[pl.ds(base, ROWS_PER_W)], idx_v)
    pltpu.async_copy(table_hbm.at[idx_v], rows_v, sem).wait()
    pltpu.sync_copy(rows_v, out_hbm.at[pl.ds(base, ROWS_PER_W)])


_sc_gather = functools.partial(
    pl.kernel,
    mesh=plsc.VectorSubcoreMesh(
        core_axis_name="c", subcore_axis_name="s", num_cores=1
    ),
    out_type=jax.ShapeDtypeStruct((N_TOK, D), jnp.float32),
    scratch_types=[
        pltpu.VMEM((ROWS_PER_W,), jnp.int32),
        pltpu.VMEM((ROWS_PER_W, D), jnp.float32),
        pltpu.SemaphoreType.DMA,
    ],
)(_sc_gather_body)


def _tc_matvec_body(w_ref, x_ref, b_ref, o_ref):
    i = pl.program_id(0)
    part = jnp.sum(w_ref[...] * x_ref[...], axis=1)

    @pl.when(i == 0)
    def _():
        o_ref[...] = part + b_ref[...]

    @pl.when(i > 0)
    def _():
        o_ref[...] = o_ref[...] + part

    @pl.when(i == N_BLKS - 1)
    def _():
        acc = o_ref[...]
        o_ref[...] = 1.0 / (1.0 + jnp.exp(-acc))


_tc_matvec = pl.pallas_call(
    _tc_matvec_body,
    grid=(N_BLKS,),
    in_specs=[
        pl.BlockSpec((FC_OUT, K_BLK), lambda i: (0, i)),
        pl.BlockSpec((1, K_BLK), lambda i: (0, i)),
        pl.BlockSpec((FC_OUT,), lambda i: (0,)),
    ],
    out_specs=pl.BlockSpec((FC_OUT,), lambda i: (0,)),
    out_shape=jax.ShapeDtypeStruct((FC_OUT,), jnp.float32),
)


@jax.jit
def kernel(text, emb_table, W, b):
    idx = text.reshape(-1)
    flat = _sc_gather(emb_table, idx)
    x = flat.reshape(1, FC_IN)
    return _tc_matvec(W, x, b)
